# baseline (speedup 1.0000x reference)
import functools

import jax
import jax.numpy as jnp
import numpy as np
from jax import lax
from jax.experimental import pallas as pl
from jax.experimental.pallas import tpu as pltpu
from jax.experimental.pallas import tpu_sc as plsc

T = 2048
H = 1024
I_DIM = 512
E = 64
K = 6
G = 8
TOPK_G = 4
C = 384
SCALE = 1.5
LIMIT = 10.0
I_S = 512

TB = 256
NW = 32
PAIRS = T * K
PPW = PAIRS // NW
PCH = 96
TPW = T // NW
TCH = 16

_NEG_INF = float("-inf")


def _roll(a, r):
    r = r % E
    if r == 0:
        return a
    return jnp.concatenate([a[:, r:], a[:, :r]], axis=1)


def _group_butterfly(a, op, lane_i):
    for s in (1, 2, 4):
        m = (lane_i & s) == 0
        partner = jnp.where(m, _roll(a, s), _roll(a, -s))
        a = op(a, partner)
    return a


def _silu(v):
    return v * (1.0 / (1.0 + jnp.exp(-v)))


def _router_body(x_ref, gw_ref, bias_ref,
                 slot_ref, w_ref, tok_ref, counts_ref, cnt_scr):
    pid = pl.program_id(0)

    @pl.when(pid == 0)
    def _():
        cnt_scr[...] = jnp.zeros((8, E), jnp.float32)

    xb = x_ref[...]
    hi = jax.lax.Precision.HIGHEST
    df = jax.lax.Precision.DEFAULT
    logits = lax.dot_general(xb, gw_ref[...], (((1,), (1,)), ((), ())),
                             precision=df, preferred_element_type=jnp.float32)
    sp = jnp.maximum(logits, 0.0) + jnp.log1p(jnp.exp(-jnp.abs(logits)))
    scores = jnp.sqrt(sp)
    s4c = scores + bias_ref[0:1, :]

    lane_f = lax.broadcasted_iota(jnp.int32, (TB, E), 1).astype(jnp.float32)
    lane_i = lax.broadcasted_iota(jnp.int32, (1, E), 1)

    gmax1 = _group_butterfly(s4c, jnp.maximum, lane_i)
    cand = jnp.where(s4c == gmax1, lane_f, 1e9)
    first = _group_butterfly(cand, jnp.minimum, lane_i)
    s_wo = jnp.where(lane_f == first, _NEG_INF, s4c)
    gmax2 = _group_butterfly(s_wo, jnp.maximum, lane_i)
    g2 = gmax1 + gmax2

    gid_i = lane_i // (E // G)
    gid_f = gid_i.astype(jnp.float32)
    rank = jnp.zeros((TB, E), jnp.float32)
    for m in range(1, G):
        sj = _roll(g2, 8 * m)
        j_f = ((gid_i + m) % G).astype(jnp.float32)
        beats = (sj > g2) | ((sj == g2) & (j_f < gid_f))
        rank = rank + beats.astype(jnp.float32)
    masked = jnp.where(rank < TOPK_G, s4c, _NEG_INF)

    cur = masked
    idx_cols, w_cols = [], []
    for _k in range(K):
        mval = jnp.max(cur, axis=1, keepdims=True)
        cnd = jnp.where(cur == mval, lane_f, 1e9)
        am = jnp.min(cnd, axis=1, keepdims=True)
        sel = lane_f == am
        w_cols.append(jnp.sum(jnp.where(sel, scores, 0.0), axis=1,
                              keepdims=True))
        idx_cols.append(am)
        cur = jnp.where(sel, _NEG_INF, cur)
    idxs = jnp.concatenate(idx_cols, axis=1)
    ws = jnp.concatenate(w_cols, axis=1)
    wn = ws / (jnp.sum(ws, axis=1, keepdims=True) + 1e-20) * SCALE

    e_cols = [idxs[:, j:j + 1] for j in range(K)]
    iota_row = lane_i.astype(jnp.float32)
    oh = jnp.zeros((TB, E), jnp.float32)
    for j in range(K):
        oh = oh + (e_cols[j] == iota_row).astype(jnp.float32)
    r_i = lax.broadcasted_iota(jnp.int32, (TB, TB), 0)
    c_j = lax.broadcasted_iota(jnp.int32, (TB, TB), 1)
    ltri = (c_j < r_i).astype(jnp.float32)
    rowcum = lax.dot_general(ltri, oh, (((1,), (0,)), ((), ())),
                             precision=hi, preferred_element_type=jnp.float32)
    base = cnt_scr[0:1, :]
    avail = base + rowcum

    slot_cols, wf_cols = [], []
    within = [jnp.zeros((TB, 1), jnp.float32) for _ in range(K)]
    for k in range(K):
        for j in range(k):
            within[k] = within[k] + (e_cols[j] == e_cols[k]).astype(jnp.float32)
        b_k = jnp.sum(jnp.where(e_cols[k] == iota_row, avail, 0.0),
                      axis=1, keepdims=True)
        pos_k = b_k + within[k]
        keep = pos_k < C
        slot_cols.append(jnp.where(keep, e_cols[k] * C + pos_k, 0.0))
        wf_cols.append(jnp.where(keep, wn[:, k:k + 1], 0.0))
    pad = jnp.zeros((TB, 2), jnp.float32)
    slot8 = jnp.concatenate(slot_cols + [pad], axis=1)
    slot_ref[...] = slot8.astype(jnp.int32)
    wrep = [jnp.broadcast_to(wf_cols[k], (TB, 16)) for k in range(K)]
    wrep.append(jnp.zeros((TB, 32), jnp.float32))
    w_ref[...] = jnp.concatenate(wrep, axis=1)
    tok_row = (lax.broadcasted_iota(jnp.int32, (TB, 8), 0)
               + pid * TB)
    tok_ref[...] = tok_row

    new_base = base + jnp.sum(oh, axis=0, keepdims=True)
    cnt_scr[...] = jnp.broadcast_to(new_base, (8, E))
    counts_ref[...] = jnp.broadcast_to(new_base, (8, E)).astype(jnp.int32)


def _router_call(x, gate_w, bias2):
    grid = (T // TB,)
    return pl.pallas_call(
        _router_body,
        grid=grid,
        in_specs=[
            pl.BlockSpec((TB, H), lambda i: (i, 0)),
            pl.BlockSpec((E, H), lambda i: (0, 0)),
            pl.BlockSpec((8, E), lambda i: (0, 0)),
        ],
        out_specs=[
            pl.BlockSpec((TB, 8), lambda i: (i, 0)),
            pl.BlockSpec((TB, 128), lambda i: (i, 0)),
            pl.BlockSpec((TB, 8), lambda i: (i, 0)),
            pl.BlockSpec((8, E), lambda i: (0, 0)),
        ],
        out_shape=[
            jax.ShapeDtypeStruct((T, 8), jnp.int32),
            jax.ShapeDtypeStruct((T, 128), jnp.float32),
            jax.ShapeDtypeStruct((T, 8), jnp.int32),
            jax.ShapeDtypeStruct((8, E), jnp.int32),
        ],
        scratch_shapes=[pltpu.VMEM((8, E), jnp.float32)],
    )(x, gate_w, bias2)


def _shared_body(x_ref, swg_ref, swu_ref, swd_ref, shared_ref):
    df = jax.lax.Precision.DEFAULT
    xb = x_ref[...].astype(jnp.bfloat16)
    sg = lax.dot_general(xb, swg_ref[...], (((1,), (0,)), ((), ())),
                         precision=df, preferred_element_type=jnp.float32)
    su = lax.dot_general(xb, swu_ref[...], (((1,), (0,)), ((), ())),
                         precision=df, preferred_element_type=jnp.float32)
    sint = jnp.clip(_silu(sg) * su, -LIMIT, LIMIT).astype(jnp.bfloat16)
    shared_ref[...] = lax.dot_general(sint, swd_ref[...],
                                      (((1,), (0,)), ((), ())),
                                      precision=df,
                                      preferred_element_type=jnp.float32)


def _shared_call(x, sw_gate, sw_up, sw_down):
    return pl.pallas_call(
        _shared_body,
        grid=(T // TB,),
        in_specs=[
            pl.BlockSpec((TB, H), lambda i: (i, 0)),
            pl.BlockSpec((H, I_S), lambda i: (0, 0)),
            pl.BlockSpec((H, I_S), lambda i: (0, 0)),
            pl.BlockSpec((I_S, H), lambda i: (0, 0)),
        ],
        out_specs=pl.BlockSpec((TB, H), lambda i: (i, 0)),
        out_shape=jax.ShapeDtypeStruct((T, H), jnp.float32),
    )(x, sw_gate, sw_up, sw_down)


def _ffn_body(buf_ref, wg_ref, wu_ref, wd_ref, out_ref):
    df = jax.lax.Precision.DEFAULT
    rows = buf_ref[...].astype(jnp.bfloat16)
    wg = wg_ref[0]
    wu = wu_ref[0]
    wd = wd_ref[0]
    gp = lax.dot_general(rows, wg, (((1,), (0,)), ((), ())),
                         precision=df, preferred_element_type=jnp.float32)
    up = lax.dot_general(rows, wu, (((1,), (0,)), ((), ())),
                         precision=df, preferred_element_type=jnp.float32)
    inter = jnp.clip(_silu(gp) * up, -LIMIT, LIMIT).astype(jnp.bfloat16)
    out_ref[...] = lax.dot_general(inter, wd, (((1,), (0,)), ((), ())),
                                   precision=df,
                                   preferred_element_type=jnp.float32)


def _ffn_call(buf, w_gate, w_up, w_down):
    return pl.pallas_call(
        _ffn_body,
        grid=(E,),
        in_specs=[
            pl.BlockSpec((C, H), lambda e: (e, 0)),
            pl.BlockSpec((1, H, I_DIM), lambda e: (e, 0, 0)),
            pl.BlockSpec((1, H, I_DIM), lambda e: (e, 0, 0)),
            pl.BlockSpec((1, I_DIM, H), lambda e: (e, 0, 0)),
        ],
        out_specs=pl.BlockSpec((C, H), lambda e: (e, 0)),
        out_shape=jax.ShapeDtypeStruct((E * C, H), jnp.float32),
    )(buf, w_gate, w_up, w_down)


def _dispatch_body(x_hbm, tok_hbm, slot_hbm, buf_hbm,
                   tokv, slotv, rows, sem_g, sem_s):
    wid = lax.axis_index("s") * 2 + lax.axis_index("c")
    base = wid * PPW
    for c4 in range(PPW // PCH):
        b = base + c4 * PCH
        pltpu.sync_copy(tok_hbm.at[pl.ds(b, PCH)], tokv)
        pltpu.sync_copy(slot_hbm.at[pl.ds(b, PCH)], slotv)
        pltpu.async_copy(x_hbm.at[tokv], rows, sem_g).wait()
        pltpu.async_copy(rows, buf_hbm.at[slotv], sem_s).wait()


def _dispatch_call(x, tok_flat, slot_flat):
    mesh = plsc.VectorSubcoreMesh(core_axis_name="c", subcore_axis_name="s", num_cores=2, num_subcores=16)
    kern = pl.kernel(
        _dispatch_body,
        out_type=jax.ShapeDtypeStruct((E * C, H), jnp.float32),
        mesh=mesh,
        scratch_types=[
            pltpu.VMEM((PCH,), jnp.int32),
            pltpu.VMEM((PCH,), jnp.int32),
            pltpu.VMEM((PCH, H), jnp.float32),
            pltpu.SemaphoreType.DMA,
            pltpu.SemaphoreType.DMA,
        ],
    )
    return kern(x, tok_flat, slot_flat)


def _combine_body(eout_hbm, slot_hbm, w_hbm, shared_hbm, out_hbm,
                  slotv, wv, rows, acc, sem_g):
    wid = lax.axis_index("s") * 2 + lax.axis_index("c")
    for c4 in range(TPW // TCH):
        pb = wid * PPW + c4 * PCH
        tb = wid * TPW + c4 * TCH
        pltpu.sync_copy(slot_hbm.at[pl.ds(pb, PCH)], slotv)
        pltpu.sync_copy(w_hbm.at[pl.ds(tb, TCH)], wv)
        pltpu.async_copy(eout_hbm.at[slotv], rows, sem_g).wait()
        pltpu.sync_copy(shared_hbm.at[pl.ds(tb, TCH)], acc)
        for tk in range(TCH):
            wspl = [wv[tk, pl.ds(k * 16, 16)] for k in range(K)]

            def col_body(c, _, tk=tk, wspl=wspl):
                a = acc[tk, pl.ds(c * 16, 16)]
                for k in range(K):
                    r = rows[tk * K + k, pl.ds(c * 16, 16)]
                    contrib = jnp.where(wspl[k] != 0.0, wspl[k] * r, 0.0)
                    a = a + contrib
                acc[tk, pl.ds(c * 16, 16)] = a
                return 0

            lax.fori_loop(0, H // 16, col_body, 0)
        pltpu.sync_copy(acc, out_hbm.at[pl.ds(tb, TCH)])


def _combine_call(eout, slot_flat, w_rep, shared):
    mesh = plsc.VectorSubcoreMesh(core_axis_name="c", subcore_axis_name="s", num_cores=2, num_subcores=16)
    kern = pl.kernel(
        _combine_body,
        out_type=jax.ShapeDtypeStruct((T, H), jnp.float32),
        mesh=mesh,
        scratch_types=[
            pltpu.VMEM((PCH,), jnp.int32),
            pltpu.VMEM((TCH, 128), jnp.float32),
            pltpu.VMEM((PCH, H), jnp.float32),
            pltpu.VMEM((TCH, H), jnp.float32),
            pltpu.SemaphoreType.DMA,
        ],
    )
    return kern(eout, slot_flat, w_rep, shared)


def kernel(x, gate_w, bias, w_gate, w_up, w_down, sw_gate, sw_up, sw_down):
    bias2 = jnp.broadcast_to(bias.reshape(1, E), (8, E))
    bf = jnp.bfloat16
    slot8, w_rep, tok8, _counts = _router_call(x, gate_w, bias2)
    slot_flat = slot8[:, :K].reshape(-1)
    tok_flat = tok8[:, :K].reshape(-1)
    buf = _dispatch_call(x, tok_flat, slot_flat)
    shared = _shared_call(x, sw_gate.astype(bf), sw_up.astype(bf),
                          sw_down.astype(bf))
    eout = _ffn_call(buf, w_gate.astype(bf), w_up.astype(bf),
                     w_down.astype(bf))
    out = _combine_call(eout, slot_flat, w_rep, shared)
    return out


# speedup vs baseline: 1.3640x; 1.3640x over previous
import functools

import jax
import jax.numpy as jnp
import numpy as np
from jax import lax
from jax.experimental import pallas as pl
from jax.experimental.pallas import tpu as pltpu
from jax.experimental.pallas import tpu_sc as plsc

T = 2048
H = 1024
I_DIM = 512
E = 64
K = 6
G = 8
TOPK_G = 4
C = 384
SCALE = 1.5
LIMIT = 10.0
I_S = 512

TB = 256
NW = 32
PAIRS = T * K
PPW = PAIRS // NW
PCH = 96
TPW = T // NW
TCH = 16

_NEG_INF = float("-inf")


def _roll(a, r):
    r = r % E
    if r == 0:
        return a
    return jnp.concatenate([a[:, r:], a[:, :r]], axis=1)


def _group_butterfly(a, op, lane_i):
    for s in (1, 2, 4):
        m = (lane_i & s) == 0
        partner = jnp.where(m, _roll(a, s), _roll(a, -s))
        a = op(a, partner)
    return a


def _silu(v):
    return v * (1.0 / (1.0 + jnp.exp(-v)))


def _router_body(x_ref, gw_ref, bias_ref,
                 slot_ref, w_ref, tok_ref, counts_ref, cnt_scr):
    pid = pl.program_id(0)

    @pl.when(pid == 0)
    def _():
        cnt_scr[...] = jnp.zeros((8, E), jnp.float32)

    xb = x_ref[...]
    hi = jax.lax.Precision.HIGHEST
    df = jax.lax.Precision.DEFAULT
    logits = lax.dot_general(xb, gw_ref[...], (((1,), (1,)), ((), ())),
                             precision=df, preferred_element_type=jnp.float32)
    sp = jnp.maximum(logits, 0.0) + jnp.log1p(jnp.exp(-jnp.abs(logits)))
    scores = jnp.sqrt(sp)
    s4c = scores + bias_ref[0:1, :]

    lane_f = lax.broadcasted_iota(jnp.int32, (TB, E), 1).astype(jnp.float32)
    lane_i = lax.broadcasted_iota(jnp.int32, (1, E), 1)

    gmax1 = _group_butterfly(s4c, jnp.maximum, lane_i)
    cand = jnp.where(s4c == gmax1, lane_f, 1e9)
    first = _group_butterfly(cand, jnp.minimum, lane_i)
    s_wo = jnp.where(lane_f == first, _NEG_INF, s4c)
    gmax2 = _group_butterfly(s_wo, jnp.maximum, lane_i)
    g2 = gmax1 + gmax2

    gid_i = lane_i // (E // G)
    gid_f = gid_i.astype(jnp.float32)
    rank = jnp.zeros((TB, E), jnp.float32)
    for m in range(1, G):
        sj = _roll(g2, 8 * m)
        j_f = ((gid_i + m) % G).astype(jnp.float32)
        beats = (sj > g2) | ((sj == g2) & (j_f < gid_f))
        rank = rank + beats.astype(jnp.float32)
    masked = jnp.where(rank < TOPK_G, s4c, _NEG_INF)

    cur = masked
    idx_cols, w_cols = [], []
    for _k in range(K):
        mval = jnp.max(cur, axis=1, keepdims=True)
        cnd = jnp.where(cur == mval, lane_f, 1e9)
        am = jnp.min(cnd, axis=1, keepdims=True)
        sel = lane_f == am
        w_cols.append(jnp.sum(jnp.where(sel, scores, 0.0), axis=1,
                              keepdims=True))
        idx_cols.append(am)
        cur = jnp.where(sel, _NEG_INF, cur)
    idxs = jnp.concatenate(idx_cols, axis=1)
    ws = jnp.concatenate(w_cols, axis=1)
    wn = ws / (jnp.sum(ws, axis=1, keepdims=True) + 1e-20) * SCALE

    e_cols = [idxs[:, j:j + 1] for j in range(K)]
    iota_row = lane_i.astype(jnp.float32)
    oh = jnp.zeros((TB, E), jnp.float32)
    for j in range(K):
        oh = oh + (e_cols[j] == iota_row).astype(jnp.float32)
    r_i = lax.broadcasted_iota(jnp.int32, (TB, TB), 0)
    c_j = lax.broadcasted_iota(jnp.int32, (TB, TB), 1)
    ltri = (c_j < r_i).astype(jnp.float32)
    rowcum = lax.dot_general(ltri, oh, (((1,), (0,)), ((), ())),
                             precision=hi, preferred_element_type=jnp.float32)
    base = cnt_scr[0:1, :]
    avail = base + rowcum

    slot_cols, wf_cols = [], []
    within = [jnp.zeros((TB, 1), jnp.float32) for _ in range(K)]
    for k in range(K):
        for j in range(k):
            within[k] = within[k] + (e_cols[j] == e_cols[k]).astype(jnp.float32)
        b_k = jnp.sum(jnp.where(e_cols[k] == iota_row, avail, 0.0),
                      axis=1, keepdims=True)
        pos_k = b_k + within[k]
        keep = pos_k < C
        slot_cols.append(jnp.where(keep, e_cols[k] * C + pos_k, 0.0))
        wf_cols.append(jnp.where(keep, wn[:, k:k + 1], 0.0))
    pad = jnp.zeros((TB, 2), jnp.float32)
    slot8 = jnp.concatenate(slot_cols + [pad], axis=1)
    slot_ref[...] = slot8.astype(jnp.int32)
    wrep = [jnp.broadcast_to(wf_cols[k], (TB, 16)) for k in range(K)]
    wrep.append(jnp.zeros((TB, 32), jnp.float32))
    w_ref[...] = jnp.concatenate(wrep, axis=1)
    tok_row = (lax.broadcasted_iota(jnp.int32, (TB, 8), 0)
               + pid * TB)
    tok_ref[...] = tok_row

    new_base = base + jnp.sum(oh, axis=0, keepdims=True)
    cnt_scr[...] = jnp.broadcast_to(new_base, (8, E))
    counts_ref[...] = jnp.broadcast_to(new_base, (8, E)).astype(jnp.int32)


def _router_call(x, gate_w, bias2):
    grid = (T // TB,)
    return pl.pallas_call(
        _router_body,
        grid=grid,
        in_specs=[
            pl.BlockSpec((TB, H), lambda i: (i, 0)),
            pl.BlockSpec((E, H), lambda i: (0, 0)),
            pl.BlockSpec((8, E), lambda i: (0, 0)),
        ],
        out_specs=[
            pl.BlockSpec((TB, 8), lambda i: (i, 0)),
            pl.BlockSpec((TB, 128), lambda i: (i, 0)),
            pl.BlockSpec((TB, 8), lambda i: (i, 0)),
            pl.BlockSpec((8, E), lambda i: (0, 0)),
        ],
        out_shape=[
            jax.ShapeDtypeStruct((T, 8), jnp.int32),
            jax.ShapeDtypeStruct((T, 128), jnp.float32),
            jax.ShapeDtypeStruct((T, 8), jnp.int32),
            jax.ShapeDtypeStruct((8, E), jnp.int32),
        ],
        scratch_shapes=[pltpu.VMEM((8, E), jnp.float32)],
    )(x, gate_w, bias2)


def _shared_body(x_ref, swg_ref, swu_ref, swd_ref, shared_ref):
    df = jax.lax.Precision.DEFAULT
    xb = x_ref[...]
    sg = lax.dot_general(xb, swg_ref[...], (((1,), (0,)), ((), ())),
                         precision=df, preferred_element_type=jnp.float32)
    su = lax.dot_general(xb, swu_ref[...], (((1,), (0,)), ((), ())),
                         precision=df, preferred_element_type=jnp.float32)
    sint = jnp.clip(_silu(sg) * su, -LIMIT, LIMIT)
    shared_ref[...] = lax.dot_general(sint, swd_ref[...],
                                      (((1,), (0,)), ((), ())),
                                      precision=df,
                                      preferred_element_type=jnp.float32)


def _shared_call(x, sw_gate, sw_up, sw_down):
    return pl.pallas_call(
        _shared_body,
        grid=(T // TB,),
        in_specs=[
            pl.BlockSpec((TB, H), lambda i: (i, 0)),
            pl.BlockSpec((H, I_S), lambda i: (0, 0)),
            pl.BlockSpec((H, I_S), lambda i: (0, 0)),
            pl.BlockSpec((I_S, H), lambda i: (0, 0)),
        ],
        out_specs=pl.BlockSpec((TB, H), lambda i: (i, 0)),
        out_shape=jax.ShapeDtypeStruct((T, H), jnp.float32),
    )(x, sw_gate, sw_up, sw_down)


def _ffn_body(buf_ref, wg_ref, wu_ref, wd_ref, out_ref):
    df = jax.lax.Precision.DEFAULT
    rows = buf_ref[...]
    wg = wg_ref[0]
    wu = wu_ref[0]
    wd = wd_ref[0]
    gp = lax.dot_general(rows, wg, (((1,), (0,)), ((), ())),
                         precision=df, preferred_element_type=jnp.float32)
    up = lax.dot_general(rows, wu, (((1,), (0,)), ((), ())),
                         precision=df, preferred_element_type=jnp.float32)
    inter = jnp.clip(_silu(gp) * up, -LIMIT, LIMIT)
    out_ref[...] = lax.dot_general(inter, wd, (((1,), (0,)), ((), ())),
                                   precision=df,
                                   preferred_element_type=jnp.float32)


def _ffn_call(buf, w_gate, w_up, w_down):
    return pl.pallas_call(
        _ffn_body,
        grid=(E,),
        in_specs=[
            pl.BlockSpec((C, H), lambda e: (e, 0)),
            pl.BlockSpec((1, H, I_DIM), lambda e: (e, 0, 0)),
            pl.BlockSpec((1, H, I_DIM), lambda e: (e, 0, 0)),
            pl.BlockSpec((1, I_DIM, H), lambda e: (e, 0, 0)),
        ],
        out_specs=pl.BlockSpec((C, H), lambda e: (e, 0)),
        out_shape=jax.ShapeDtypeStruct((E * C, H), jnp.float32),
    )(buf, w_gate, w_up, w_down)


def _dispatch_body(x_hbm, tok_hbm, slot_hbm, buf_hbm,
                   tokv, slotv, rows, sem_g, sem_s):
    wid = lax.axis_index("s") * 2 + lax.axis_index("c")
    base = wid * PPW
    for c4 in range(PPW // PCH):
        b = base + c4 * PCH
        pltpu.sync_copy(tok_hbm.at[pl.ds(b, PCH)], tokv)
        pltpu.sync_copy(slot_hbm.at[pl.ds(b, PCH)], slotv)
        pltpu.async_copy(x_hbm.at[tokv], rows, sem_g).wait()
        pltpu.async_copy(rows, buf_hbm.at[slotv], sem_s).wait()


def _dispatch_call(x, tok_flat, slot_flat):
    mesh = plsc.VectorSubcoreMesh(core_axis_name="c", subcore_axis_name="s", num_cores=2, num_subcores=16)
    kern = pl.kernel(
        _dispatch_body,
        out_type=jax.ShapeDtypeStruct((E * C, H), jnp.float32),
        mesh=mesh,
        scratch_types=[
            pltpu.VMEM((PCH,), jnp.int32),
            pltpu.VMEM((PCH,), jnp.int32),
            pltpu.VMEM((PCH, H), jnp.float32),
            pltpu.SemaphoreType.DMA,
            pltpu.SemaphoreType.DMA,
        ],
    )
    return kern(x, tok_flat, slot_flat)


def _combine_body(eout_hbm, slot_hbm, w_hbm, shared_hbm, out_hbm,
                  slotv, wv, rows, acc, sem_g):
    wid = lax.axis_index("s") * 2 + lax.axis_index("c")
    for c4 in range(TPW // TCH):
        pb = wid * PPW + c4 * PCH
        tb = wid * TPW + c4 * TCH
        pltpu.sync_copy(slot_hbm.at[pl.ds(pb, PCH)], slotv)
        pltpu.sync_copy(w_hbm.at[pl.ds(tb, TCH)], wv)
        pltpu.async_copy(eout_hbm.at[slotv], rows, sem_g).wait()
        pltpu.sync_copy(shared_hbm.at[pl.ds(tb, TCH)], acc)
        for tk in range(TCH):
            wspl = [wv[tk, pl.ds(k * 16, 16)] for k in range(K)]

            def col_body(c, _, tk=tk, wspl=wspl):
                a = acc[tk, pl.ds(c * 16, 16)]
                for k in range(K):
                    r = rows[tk * K + k, pl.ds(c * 16, 16)]
                    contrib = jnp.where(wspl[k] != 0.0, wspl[k] * r, 0.0)
                    a = a + contrib
                acc[tk, pl.ds(c * 16, 16)] = a
                return 0

            lax.fori_loop(0, H // 16, col_body, 0)
        pltpu.sync_copy(acc, out_hbm.at[pl.ds(tb, TCH)])


def _combine_call(eout, slot_flat, w_rep, shared):
    mesh = plsc.VectorSubcoreMesh(core_axis_name="c", subcore_axis_name="s", num_cores=2, num_subcores=16)
    kern = pl.kernel(
        _combine_body,
        out_type=jax.ShapeDtypeStruct((T, H), jnp.float32),
        mesh=mesh,
        scratch_types=[
            pltpu.VMEM((PCH,), jnp.int32),
            pltpu.VMEM((TCH, 128), jnp.float32),
            pltpu.VMEM((PCH, H), jnp.float32),
            pltpu.VMEM((TCH, H), jnp.float32),
            pltpu.SemaphoreType.DMA,
        ],
    )
    return kern(eout, slot_flat, w_rep, shared)


def kernel(x, gate_w, bias, w_gate, w_up, w_down, sw_gate, sw_up, sw_down):
    bias2 = jnp.broadcast_to(bias.reshape(1, E), (8, E))
    slot8, w_rep, tok8, _counts = _router_call(x, gate_w, bias2)
    slot_flat = slot8[:, :K].reshape(-1)
    tok_flat = tok8[:, :K].reshape(-1)
    buf = _dispatch_call(x, tok_flat, slot_flat)
    shared = _shared_call(x, sw_gate, sw_up, sw_down)
    eout = _ffn_call(buf, w_gate, w_up, w_down)
    out = _combine_call(eout, slot_flat, w_rep, shared)
    return out


# speedup vs baseline: 1.4742x; 1.0808x over previous
import functools

import jax
import jax.numpy as jnp
import numpy as np
from jax import lax
from jax.experimental import pallas as pl
from jax.experimental.pallas import tpu as pltpu
from jax.experimental.pallas import tpu_sc as plsc

T = 2048
H = 1024
I_DIM = 512
E = 64
K = 6
G = 8
TOPK_G = 4
C = 384
SCALE = 1.5
LIMIT = 10.0
I_S = 512

TB = 256
NW = 32
PAIRS = T * K
PPW = PAIRS // NW
PCH = 96
TPW = T // NW
TCH = 16

_NEG_INF = float("-inf")


def _roll(a, r):
    r = r % E
    if r == 0:
        return a
    return jnp.concatenate([a[:, r:], a[:, :r]], axis=1)


def _group_butterfly(a, op, lane_i):
    for s in (1, 2, 4):
        m = (lane_i & s) == 0
        partner = jnp.where(m, _roll(a, s), _roll(a, -s))
        a = op(a, partner)
    return a


def _silu(v):
    return v * (1.0 / (1.0 + jnp.exp(-v)))


def _router_body(x_ref, gw_ref, bias_ref, swg_ref, swu_ref, swd_ref,
                 shared_ref, slot_ref, slott_ref, w_ref, counts_ref,
                 cnt_scr):
    pid = pl.program_id(0)

    @pl.when(pid == 0)
    def _():
        cnt_scr[...] = jnp.zeros((8, E), jnp.float32)

    xb = x_ref[...]
    hi = jax.lax.Precision.HIGHEST
    df = jax.lax.Precision.DEFAULT
    logits = lax.dot_general(xb, gw_ref[...], (((1,), (1,)), ((), ())),
                             precision=df, preferred_element_type=jnp.float32)
    sp = jnp.maximum(logits, 0.0) + jnp.log1p(jnp.exp(-jnp.abs(logits)))
    scores = jnp.sqrt(sp)
    s4c = scores + bias_ref[0:1, :]

    lane_f = lax.broadcasted_iota(jnp.int32, (TB, E), 1).astype(jnp.float32)
    lane_i = lax.broadcasted_iota(jnp.int32, (1, E), 1)

    gmax1 = _group_butterfly(s4c, jnp.maximum, lane_i)
    cand = jnp.where(s4c == gmax1, lane_f, 1e9)
    first = _group_butterfly(cand, jnp.minimum, lane_i)
    s_wo = jnp.where(lane_f == first, _NEG_INF, s4c)
    gmax2 = _group_butterfly(s_wo, jnp.maximum, lane_i)
    g2 = gmax1 + gmax2

    gid_i = lane_i // (E // G)
    gid_f = gid_i.astype(jnp.float32)
    rank = jnp.zeros((TB, E), jnp.float32)
    for m in range(1, G):
        sj = _roll(g2, 8 * m)
        j_f = ((gid_i + m) % G).astype(jnp.float32)
        beats = (sj > g2) | ((sj == g2) & (j_f < gid_f))
        rank = rank + beats.astype(jnp.float32)
    masked = jnp.where(rank < TOPK_G, s4c, _NEG_INF)

    cur = masked
    idx_cols, w_cols = [], []
    for _k in range(K):
        mval = jnp.max(cur, axis=1, keepdims=True)
        cnd = jnp.where(cur == mval, lane_f, 1e9)
        am = jnp.min(cnd, axis=1, keepdims=True)
        sel = lane_f == am
        w_cols.append(jnp.sum(jnp.where(sel, scores, 0.0), axis=1,
                              keepdims=True))
        idx_cols.append(am)
        cur = jnp.where(sel, _NEG_INF, cur)
    idxs = jnp.concatenate(idx_cols, axis=1)
    ws = jnp.concatenate(w_cols, axis=1)
    wn = ws / (jnp.sum(ws, axis=1, keepdims=True) + 1e-20) * SCALE

    e_cols = [idxs[:, j:j + 1] for j in range(K)]
    iota_row = lane_i.astype(jnp.float32)
    oh = jnp.zeros((TB, E), jnp.float32)
    for j in range(K):
        oh = oh + (e_cols[j] == iota_row).astype(jnp.float32)
    r_i = lax.broadcasted_iota(jnp.int32, (TB, TB), 0)
    c_j = lax.broadcasted_iota(jnp.int32, (TB, TB), 1)
    ltri = (c_j < r_i).astype(jnp.float32)
    rowcum = lax.dot_general(ltri, oh, (((1,), (0,)), ((), ())),
                             precision=hi, preferred_element_type=jnp.float32)
    base = cnt_scr[0:1, :]
    avail = base + rowcum

    slot_cols, wf_cols = [], []
    within = [jnp.zeros((TB, 1), jnp.float32) for _ in range(K)]
    for k in range(K):
        for j in range(k):
            within[k] = within[k] + (e_cols[j] == e_cols[k]).astype(jnp.float32)
        b_k = jnp.sum(jnp.where(e_cols[k] == iota_row, avail, 0.0),
                      axis=1, keepdims=True)
        pos_k = b_k + within[k]
        keep = pos_k < C
        slot_cols.append(jnp.where(keep, e_cols[k] * C + pos_k, 0.0))
        wf_cols.append(jnp.where(keep, wn[:, k:k + 1], 0.0))
    pad = jnp.zeros((TB, 2), jnp.float32)
    slot8 = jnp.concatenate(slot_cols + [pad], axis=1)
    slot_ref[...] = slot8.astype(jnp.int32)
    slott_ref[...] = jnp.transpose(slot8, (1, 0)).astype(jnp.int32)
    wrep = [jnp.broadcast_to(wf_cols[k], (TB, 16)) for k in range(K)]
    wrep.append(jnp.zeros((TB, 32), jnp.float32))
    w_ref[...] = jnp.concatenate(wrep, axis=1)

    new_base = base + jnp.sum(oh, axis=0, keepdims=True)
    cnt_scr[...] = jnp.broadcast_to(new_base, (8, E))
    counts_ref[...] = jnp.broadcast_to(new_base, (8, E)).astype(jnp.int32)

    sg = lax.dot_general(xb, swg_ref[...], (((1,), (0,)), ((), ())),
                         precision=df, preferred_element_type=jnp.float32)
    su = lax.dot_general(xb, swu_ref[...], (((1,), (0,)), ((), ())),
                         precision=df, preferred_element_type=jnp.float32)
    sint = jnp.clip(_silu(sg) * su, -LIMIT, LIMIT)
    shared_ref[...] = lax.dot_general(sint, swd_ref[...],
                                      (((1,), (0,)), ((), ())),
                                      precision=df,
                                      preferred_element_type=jnp.float32)


def _router_call(x, gate_w, bias2, sw_gate, sw_up, sw_down):
    grid = (T // TB,)
    return pl.pallas_call(
        _router_body,
        grid=grid,
        in_specs=[
            pl.BlockSpec((TB, H), lambda i: (i, 0)),
            pl.BlockSpec((E, H), lambda i: (0, 0)),
            pl.BlockSpec((8, E), lambda i: (0, 0)),
            pl.BlockSpec((H, I_S), lambda i: (0, 0)),
            pl.BlockSpec((H, I_S), lambda i: (0, 0)),
            pl.BlockSpec((I_S, H), lambda i: (0, 0)),
        ],
        out_specs=[
            pl.BlockSpec((TB, H), lambda i: (i, 0)),
            pl.BlockSpec((TB, 8), lambda i: (i, 0)),
            pl.BlockSpec((8, TB), lambda i: (0, i)),
            pl.BlockSpec((TB, 128), lambda i: (i, 0)),
            pl.BlockSpec((8, E), lambda i: (0, 0)),
        ],
        out_shape=[
            jax.ShapeDtypeStruct((T, H), jnp.float32),
            jax.ShapeDtypeStruct((T, 8), jnp.int32),
            jax.ShapeDtypeStruct((8, T), jnp.int32),
            jax.ShapeDtypeStruct((T, 128), jnp.float32),
            jax.ShapeDtypeStruct((8, E), jnp.int32),
        ],
        scratch_shapes=[pltpu.VMEM((8, E), jnp.float32)],
    )(x, gate_w, bias2, sw_gate, sw_up, sw_down)


def _ffn_body(buf_ref, wg_ref, wu_ref, wd_ref, out_ref):
    df = jax.lax.Precision.DEFAULT
    rows = buf_ref[...]
    wg = wg_ref[0]
    wu = wu_ref[0]
    wd = wd_ref[0]
    gp = lax.dot_general(rows, wg, (((1,), (0,)), ((), ())),
                         precision=df, preferred_element_type=jnp.float32)
    up = lax.dot_general(rows, wu, (((1,), (0,)), ((), ())),
                         precision=df, preferred_element_type=jnp.float32)
    inter = jnp.clip(_silu(gp) * up, -LIMIT, LIMIT)
    out_ref[...] = lax.dot_general(inter, wd, (((1,), (0,)), ((), ())),
                                   precision=df,
                                   preferred_element_type=jnp.float32)


def _ffn_call(buf, w_gate, w_up, w_down):
    return pl.pallas_call(
        _ffn_body,
        grid=(E,),
        in_specs=[
            pl.BlockSpec((C, H), lambda e: (e, 0)),
            pl.BlockSpec((1, H, I_DIM), lambda e: (e, 0, 0)),
            pl.BlockSpec((1, H, I_DIM), lambda e: (e, 0, 0)),
            pl.BlockSpec((1, I_DIM, H), lambda e: (e, 0, 0)),
        ],
        out_specs=pl.BlockSpec((C, H), lambda e: (e, 0)),
        out_shape=jax.ShapeDtypeStruct((E * C, H), jnp.float32),
    )(buf, w_gate, w_up, w_down)


DTCH = 32


def _dispatch_body(x_hbm, slott_hbm, buf_hbm,
                   rows, idxs, sem_x, sem_s):
    wid = lax.axis_index("s") * 2 + lax.axis_index("c")
    tbase = wid * TPW
    for c2 in range(TPW // DTCH):
        tb = tbase + c2 * DTCH
        pltpu.sync_copy(x_hbm.at[pl.ds(tb, DTCH)], rows)
        for k in range(K):
            pltpu.sync_copy(slott_hbm.at[k, pl.ds(tb, DTCH)], idxs[k])
        cps = [pltpu.async_copy(rows, buf_hbm.at[idxs[k]], sem_s)
               for k in range(K)]
        for cp in cps:
            cp.wait()


def _dispatch_call(x, slott):
    mesh = plsc.VectorSubcoreMesh(core_axis_name="c", subcore_axis_name="s", num_cores=2, num_subcores=16)
    kern = pl.kernel(
        _dispatch_body,
        out_type=jax.ShapeDtypeStruct((E * C, H), jnp.float32),
        mesh=mesh,
        scratch_types=[
            pltpu.VMEM((DTCH, H), jnp.float32),
            [pltpu.VMEM((DTCH,), jnp.int32) for _ in range(K)],
            pltpu.SemaphoreType.DMA,
            pltpu.SemaphoreType.DMA,
        ],
    )
    return kern(x, slott)


def _combine_body(eout_hbm, slot_hbm, w_hbm, shared_hbm, out_hbm,
                  slotv, wv, rows, acc, sem_g):
    wid = lax.axis_index("s") * 2 + lax.axis_index("c")
    for c4 in range(TPW // TCH):
        pb = wid * PPW + c4 * PCH
        tb = wid * TPW + c4 * TCH
        pltpu.sync_copy(slot_hbm.at[pl.ds(pb, PCH)], slotv)
        pltpu.sync_copy(w_hbm.at[pl.ds(tb, TCH)], wv)
        pltpu.async_copy(eout_hbm.at[slotv], rows, sem_g).wait()
        pltpu.sync_copy(shared_hbm.at[pl.ds(tb, TCH)], acc)
        for tk in range(TCH):
            wspl = [wv[tk, pl.ds(k * 16, 16)] for k in range(K)]

            def col_body(c, _, tk=tk, wspl=wspl):
                a = acc[tk, pl.ds(c * 16, 16)]
                for k in range(K):
                    r = rows[tk * K + k, pl.ds(c * 16, 16)]
                    contrib = jnp.where(wspl[k] != 0.0, wspl[k] * r, 0.0)
                    a = a + contrib
                acc[tk, pl.ds(c * 16, 16)] = a
                return 0

            lax.fori_loop(0, H // 16, col_body, 0)
        pltpu.sync_copy(acc, out_hbm.at[pl.ds(tb, TCH)])


def _combine_call(eout, slot_flat, w_rep, shared):
    mesh = plsc.VectorSubcoreMesh(core_axis_name="c", subcore_axis_name="s", num_cores=2, num_subcores=16)
    kern = pl.kernel(
        _combine_body,
        out_type=jax.ShapeDtypeStruct((T, H), jnp.float32),
        mesh=mesh,
        scratch_types=[
            pltpu.VMEM((PCH,), jnp.int32),
            pltpu.VMEM((TCH, 128), jnp.float32),
            pltpu.VMEM((PCH, H), jnp.float32),
            pltpu.VMEM((TCH, H), jnp.float32),
            pltpu.SemaphoreType.DMA,
        ],
    )
    return kern(eout, slot_flat, w_rep, shared)


def kernel(x, gate_w, bias, w_gate, w_up, w_down, sw_gate, sw_up, sw_down):
    bias2 = jnp.broadcast_to(bias.reshape(1, E), (8, E))
    shared, slot8, slott, w_rep, _counts = _router_call(
        x, gate_w, bias2, sw_gate, sw_up, sw_down)
    slot_flat = slot8[:, :K].reshape(-1)
    buf = _dispatch_call(x, slott)
    eout = _ffn_call(buf, w_gate, w_up, w_down)
    out = _combine_call(eout, slot_flat, w_rep, shared)
    return out


# speedup vs baseline: 1.5594x; 1.0578x over previous
import functools

import jax
import jax.numpy as jnp
import numpy as np
from jax import lax
from jax.experimental import pallas as pl
from jax.experimental.pallas import tpu as pltpu
from jax.experimental.pallas import tpu_sc as plsc

T = 2048
H = 1024
I_DIM = 512
E = 64
K = 6
G = 8
TOPK_G = 4
C = 384
SCALE = 1.5
LIMIT = 10.0
I_S = 512

TB = 256
NW = 32
PAIRS = T * K
PPW = PAIRS // NW
PCH = 96
TPW = T // NW
TCH = 16

_NEG_INF = float("-inf")


def _roll(a, r):
    r = r % E
    if r == 0:
        return a
    return jnp.concatenate([a[:, r:], a[:, :r]], axis=1)


def _group_butterfly(a, op, lane_i):
    for s in (1, 2, 4):
        m = (lane_i & s) == 0
        partner = jnp.where(m, _roll(a, s), _roll(a, -s))
        a = op(a, partner)
    return a


def _silu(v):
    return v * (1.0 / (1.0 + jnp.exp(-v)))


def _router_body(x_ref, gw_ref, bias_ref, swg_ref, swu_ref, swd_ref,
                 shared_ref, slot_ref, slott_ref, w_ref, counts_ref,
                 cnt_scr):
    pid = pl.program_id(0)

    @pl.when(pid == 0)
    def _():
        cnt_scr[...] = jnp.zeros((8, E), jnp.float32)

    xb = x_ref[...]
    hi = jax.lax.Precision.HIGHEST
    df = jax.lax.Precision.DEFAULT
    logits = lax.dot_general(xb, gw_ref[...], (((1,), (1,)), ((), ())),
                             precision=df, preferred_element_type=jnp.float32)
    sp = jnp.maximum(logits, 0.0) + jnp.log1p(jnp.exp(-jnp.abs(logits)))
    scores = jnp.sqrt(sp)
    s4c = scores + bias_ref[0:1, :]

    lane_f = lax.broadcasted_iota(jnp.int32, (TB, E), 1).astype(jnp.float32)
    lane_i = lax.broadcasted_iota(jnp.int32, (1, E), 1)

    gmax1 = _group_butterfly(s4c, jnp.maximum, lane_i)
    cand = jnp.where(s4c == gmax1, lane_f, 1e9)
    first = _group_butterfly(cand, jnp.minimum, lane_i)
    s_wo = jnp.where(lane_f == first, _NEG_INF, s4c)
    gmax2 = _group_butterfly(s_wo, jnp.maximum, lane_i)
    g2 = gmax1 + gmax2

    gid_i = lane_i // (E // G)
    gid_f = gid_i.astype(jnp.float32)
    rank = jnp.zeros((TB, E), jnp.float32)
    for m in range(1, G):
        sj = _roll(g2, 8 * m)
        j_f = ((gid_i + m) % G).astype(jnp.float32)
        beats = (sj > g2) | ((sj == g2) & (j_f < gid_f))
        rank = rank + beats.astype(jnp.float32)
    masked = jnp.where(rank < TOPK_G, s4c, _NEG_INF)

    cur = masked
    idx_cols, w_cols = [], []
    for _k in range(K):
        mval = jnp.max(cur, axis=1, keepdims=True)
        cnd = jnp.where(cur == mval, lane_f, 1e9)
        am = jnp.min(cnd, axis=1, keepdims=True)
        sel = lane_f == am
        w_cols.append(jnp.sum(jnp.where(sel, scores, 0.0), axis=1,
                              keepdims=True))
        idx_cols.append(am)
        cur = jnp.where(sel, _NEG_INF, cur)
    idxs = jnp.concatenate(idx_cols, axis=1)
    ws = jnp.concatenate(w_cols, axis=1)
    wn = ws / (jnp.sum(ws, axis=1, keepdims=True) + 1e-20) * SCALE

    e_cols = [idxs[:, j:j + 1] for j in range(K)]
    iota_row = lane_i.astype(jnp.float32)
    oh = jnp.zeros((TB, E), jnp.float32)
    for j in range(K):
        oh = oh + (e_cols[j] == iota_row).astype(jnp.float32)
    r_i = lax.broadcasted_iota(jnp.int32, (TB, TB), 0)
    c_j = lax.broadcasted_iota(jnp.int32, (TB, TB), 1)
    ltri = (c_j < r_i).astype(jnp.float32)
    rowcum = lax.dot_general(ltri, oh, (((1,), (0,)), ((), ())),
                             precision=hi, preferred_element_type=jnp.float32)
    base = cnt_scr[0:1, :]
    avail = base + rowcum

    slot_cols, wf_cols = [], []
    within = [jnp.zeros((TB, 1), jnp.float32) for _ in range(K)]
    for k in range(K):
        for j in range(k):
            within[k] = within[k] + (e_cols[j] == e_cols[k]).astype(jnp.float32)
        b_k = jnp.sum(jnp.where(e_cols[k] == iota_row, avail, 0.0),
                      axis=1, keepdims=True)
        pos_k = b_k + within[k]
        keep = pos_k < C
        slot_cols.append(jnp.where(keep, e_cols[k] * C + pos_k, 0.0))
        wf_cols.append(jnp.where(keep, wn[:, k:k + 1], 0.0))
    pad = jnp.zeros((TB, 2), jnp.float32)
    slot8 = jnp.concatenate(slot_cols + [pad], axis=1)
    slot_ref[...] = slot8.astype(jnp.int32)
    slott_ref[...] = jnp.transpose(slot8, (1, 0)).astype(jnp.int32)
    wrep = [jnp.broadcast_to(wf_cols[k], (TB, 16)) for k in range(K)]
    wrep.append(jnp.zeros((TB, 32), jnp.float32))
    w_ref[...] = jnp.concatenate(wrep, axis=1)

    new_base = base + jnp.sum(oh, axis=0, keepdims=True)
    cnt_scr[...] = jnp.broadcast_to(new_base, (8, E))
    counts_ref[...] = jnp.broadcast_to(new_base, (8, E)).astype(jnp.int32)

    sg = lax.dot_general(xb, swg_ref[...], (((1,), (0,)), ((), ())),
                         precision=df, preferred_element_type=jnp.float32)
    su = lax.dot_general(xb, swu_ref[...], (((1,), (0,)), ((), ())),
                         precision=df, preferred_element_type=jnp.float32)
    sint = jnp.clip(_silu(sg) * su, -LIMIT, LIMIT)
    shared_ref[...] = lax.dot_general(sint, swd_ref[...],
                                      (((1,), (0,)), ((), ())),
                                      precision=df,
                                      preferred_element_type=jnp.float32)


def _router_call(x, gate_w, bias2, sw_gate, sw_up, sw_down):
    grid = (T // TB,)
    return pl.pallas_call(
        _router_body,
        grid=grid,
        in_specs=[
            pl.BlockSpec((TB, H), lambda i: (i, 0)),
            pl.BlockSpec((E, H), lambda i: (0, 0)),
            pl.BlockSpec((8, E), lambda i: (0, 0)),
            pl.BlockSpec((H, I_S), lambda i: (0, 0)),
            pl.BlockSpec((H, I_S), lambda i: (0, 0)),
            pl.BlockSpec((I_S, H), lambda i: (0, 0)),
        ],
        out_specs=[
            pl.BlockSpec((TB, H), lambda i: (i, 0)),
            pl.BlockSpec((TB, 8), lambda i: (i, 0)),
            pl.BlockSpec((8, TB), lambda i: (0, i)),
            pl.BlockSpec((TB, 128), lambda i: (i, 0)),
            pl.BlockSpec((8, E), lambda i: (0, 0)),
        ],
        out_shape=[
            jax.ShapeDtypeStruct((T, H), jnp.float32),
            jax.ShapeDtypeStruct((T, 8), jnp.int32),
            jax.ShapeDtypeStruct((8, T), jnp.int32),
            jax.ShapeDtypeStruct((T, 128), jnp.float32),
            jax.ShapeDtypeStruct((8, E), jnp.int32),
        ],
        scratch_shapes=[pltpu.VMEM((8, E), jnp.float32)],
    )(x, gate_w, bias2, sw_gate, sw_up, sw_down)


def _ffn_body(buf_ref, wg_ref, wu_ref, wd_ref, out_ref):
    df = jax.lax.Precision.DEFAULT
    rows = buf_ref[...]
    wg = wg_ref[0]
    wu = wu_ref[0]
    wd = wd_ref[0]
    gp = lax.dot_general(rows, wg, (((1,), (0,)), ((), ())),
                         precision=df, preferred_element_type=jnp.float32)
    up = lax.dot_general(rows, wu, (((1,), (0,)), ((), ())),
                         precision=df, preferred_element_type=jnp.float32)
    inter = jnp.clip(_silu(gp) * up, -LIMIT, LIMIT)
    out_ref[...] = lax.dot_general(inter, wd, (((1,), (0,)), ((), ())),
                                   precision=df,
                                   preferred_element_type=jnp.float32)


def _ffn_call(buf, w_gate, w_up, w_down):
    return pl.pallas_call(
        _ffn_body,
        grid=(E,),
        in_specs=[
            pl.BlockSpec((C, H), lambda e: (e, 0)),
            pl.BlockSpec((1, H, I_DIM), lambda e: (e, 0, 0)),
            pl.BlockSpec((1, H, I_DIM), lambda e: (e, 0, 0)),
            pl.BlockSpec((1, I_DIM, H), lambda e: (e, 0, 0)),
        ],
        out_specs=pl.BlockSpec((C, H), lambda e: (e, 0)),
        out_shape=jax.ShapeDtypeStruct((E * C, H), jnp.float32),
    )(buf, w_gate, w_up, w_down)


DTCH = 32


def _dispatch_body(x_hbm, slott_hbm, buf_hbm,
                   rows, idxs, sem_x, sem_s):
    wid = lax.axis_index("s") * 2 + lax.axis_index("c")
    tbase = wid * TPW
    for c2 in range(TPW // DTCH):
        tb = tbase + c2 * DTCH
        pltpu.sync_copy(x_hbm.at[pl.ds(tb, DTCH)], rows)
        for k in range(K):
            pltpu.sync_copy(slott_hbm.at[k, pl.ds(tb, DTCH)], idxs[k])
        cps = [pltpu.async_copy(rows, buf_hbm.at[idxs[k]], sem_s)
               for k in range(K)]
        for cp in cps:
            cp.wait()


def _dispatch_call(x, slott):
    mesh = plsc.VectorSubcoreMesh(core_axis_name="c", subcore_axis_name="s", num_cores=2, num_subcores=16)
    kern = pl.kernel(
        _dispatch_body,
        out_type=jax.ShapeDtypeStruct((E * C, H), jnp.float32),
        mesh=mesh,
        scratch_types=[
            pltpu.VMEM((DTCH, H), jnp.float32),
            [pltpu.VMEM((DTCH,), jnp.int32) for _ in range(K)],
            pltpu.SemaphoreType.DMA,
            pltpu.SemaphoreType.DMA,
        ],
    )
    return kern(x, slott)


def _combine_body(eout_hbm, slot_hbm, w_hbm, shared_hbm, out_hbm,
                  slotv, wv, rows, acc, sem_g):
    wid = lax.axis_index("s") * 2 + lax.axis_index("c")
    for c4 in range(TPW // TCH):
        pb = wid * PPW + c4 * PCH
        tb = wid * TPW + c4 * TCH
        pltpu.sync_copy(slot_hbm.at[pl.ds(pb, PCH)], slotv)
        pltpu.sync_copy(w_hbm.at[pl.ds(tb, TCH)], wv)
        pltpu.async_copy(eout_hbm.at[slotv], rows, sem_g).wait()
        pltpu.sync_copy(shared_hbm.at[pl.ds(tb, TCH)], acc)
        for tk in range(TCH):
            wspl = [wv[tk, pl.ds(k * 16, 16)] for k in range(K)]

            @plsc.parallel_loop(0, H // 16, 1, unroll=4)
            def _col_body(c, tk=tk, wspl=wspl):
                a = acc[tk, pl.ds(c * 16, 16)]
                for k in range(K):
                    r = rows[tk * K + k, pl.ds(c * 16, 16)]
                    contrib = jnp.where(wspl[k] != 0.0, wspl[k] * r, 0.0)
                    a = a + contrib
                acc[tk, pl.ds(c * 16, 16)] = a
        pltpu.sync_copy(acc, out_hbm.at[pl.ds(tb, TCH)])


def _combine_call(eout, slot_flat, w_rep, shared):
    mesh = plsc.VectorSubcoreMesh(core_axis_name="c", subcore_axis_name="s", num_cores=2, num_subcores=16)
    kern = pl.kernel(
        _combine_body,
        out_type=jax.ShapeDtypeStruct((T, H), jnp.float32),
        mesh=mesh,
        scratch_types=[
            pltpu.VMEM((PCH,), jnp.int32),
            pltpu.VMEM((TCH, 128), jnp.float32),
            pltpu.VMEM((PCH, H), jnp.float32),
            pltpu.VMEM((TCH, H), jnp.float32),
            pltpu.SemaphoreType.DMA,
        ],
    )
    return kern(eout, slot_flat, w_rep, shared)


def kernel(x, gate_w, bias, w_gate, w_up, w_down, sw_gate, sw_up, sw_down):
    bias2 = jnp.broadcast_to(bias.reshape(1, E), (8, E))
    shared, slot8, slott, w_rep, _counts = _router_call(
        x, gate_w, bias2, sw_gate, sw_up, sw_down)
    slot_flat = slot8[:, :K].reshape(-1)
    buf = _dispatch_call(x, slott)
    eout = _ffn_call(buf, w_gate, w_up, w_down)
    out = _combine_call(eout, slot_flat, w_rep, shared)
    return out


# speedup vs baseline: 1.5677x; 1.0053x over previous
import functools

import jax
import jax.numpy as jnp
import numpy as np
from jax import lax
from jax.experimental import pallas as pl
from jax.experimental.pallas import tpu as pltpu
from jax.experimental.pallas import tpu_sc as plsc

T = 2048
H = 1024
I_DIM = 512
E = 64
K = 6
G = 8
TOPK_G = 4
C = 384
SCALE = 1.5
LIMIT = 10.0
I_S = 512

TB = 256
NW = 32
PAIRS = T * K
PPW = PAIRS // NW
PCH = 96
TPW = T // NW
TCH = 16

_NEG_INF = float("-inf")


def _roll(a, r):
    r = r % E
    if r == 0:
        return a
    return jnp.concatenate([a[:, r:], a[:, :r]], axis=1)


def _group_butterfly(a, op, lane_i):
    for s in (1, 2, 4):
        m = (lane_i & s) == 0
        partner = jnp.where(m, _roll(a, s), _roll(a, -s))
        a = op(a, partner)
    return a


def _silu(v):
    return v * (1.0 / (1.0 + jnp.exp(-v)))


def _router_body(x_ref, gw_ref, bias_ref, swg_ref, swu_ref, swd_ref,
                 shared_ref, slot_ref, slott_ref, w_ref, counts_ref,
                 cnt_scr):
    pid = pl.program_id(0)

    @pl.when(pid == 0)
    def _():
        cnt_scr[...] = jnp.zeros((8, E), jnp.float32)

    xb = x_ref[...]
    hi = jax.lax.Precision.HIGHEST
    df = jax.lax.Precision.DEFAULT
    logits = lax.dot_general(xb, gw_ref[...], (((1,), (1,)), ((), ())),
                             precision=df, preferred_element_type=jnp.float32)
    sp = jnp.maximum(logits, 0.0) + jnp.log1p(jnp.exp(-jnp.abs(logits)))
    scores = jnp.sqrt(sp)
    s4c = scores + bias_ref[0:1, :]

    lane_f = lax.broadcasted_iota(jnp.int32, (TB, E), 1).astype(jnp.float32)
    lane_i = lax.broadcasted_iota(jnp.int32, (1, E), 1)

    pmax = jnp.full((TB, E), _NEG_INF, jnp.float32)
    for r in range(1, E // G):
        cand = s4c + _roll(s4c, r)
        ok = (lane_i & 7) < (8 - r)
        pmax = jnp.maximum(pmax, jnp.where(ok, cand, _NEG_INF))
    g2 = _group_butterfly(pmax, jnp.maximum, lane_i)

    gid_i = lane_i // (E // G)
    gid_f = gid_i.astype(jnp.float32)
    rank = jnp.zeros((TB, E), jnp.float32)
    for m in range(1, G):
        sj = _roll(g2, 8 * m)
        j_f = ((gid_i + m) % G).astype(jnp.float32)
        beats = (sj > g2) | ((sj == g2) & (j_f < gid_f))
        rank = rank + beats.astype(jnp.float32)
    masked = jnp.where(rank < TOPK_G, s4c, _NEG_INF)

    cur = masked
    idx_cols, w_cols = [], []
    for _k in range(K):
        mval = jnp.max(cur, axis=1, keepdims=True)
        cnd = jnp.where(cur == mval, lane_f, 1e9)
        am = jnp.min(cnd, axis=1, keepdims=True)
        sel = lane_f == am
        w_cols.append(jnp.sum(jnp.where(sel, scores, 0.0), axis=1,
                              keepdims=True))
        idx_cols.append(am)
        cur = jnp.where(sel, _NEG_INF, cur)
    idxs = jnp.concatenate(idx_cols, axis=1)
    ws = jnp.concatenate(w_cols, axis=1)
    wn = ws / (jnp.sum(ws, axis=1, keepdims=True) + 1e-20) * SCALE

    e_cols = [idxs[:, j:j + 1] for j in range(K)]
    iota_row = lane_i.astype(jnp.float32)
    oh = jnp.zeros((TB, E), jnp.float32)
    for j in range(K):
        oh = oh + (e_cols[j] == iota_row).astype(jnp.float32)
    r_i = lax.broadcasted_iota(jnp.int32, (TB, TB), 0)
    c_j = lax.broadcasted_iota(jnp.int32, (TB, TB), 1)
    ltri = (c_j < r_i).astype(jnp.float32)
    rowcum = lax.dot_general(ltri, oh, (((1,), (0,)), ((), ())),
                             precision=hi, preferred_element_type=jnp.float32)
    base = cnt_scr[0:1, :]
    avail = base + rowcum

    slot_cols, wf_cols = [], []
    within = [jnp.zeros((TB, 1), jnp.float32) for _ in range(K)]
    for k in range(K):
        for j in range(k):
            within[k] = within[k] + (e_cols[j] == e_cols[k]).astype(jnp.float32)
        b_k = jnp.sum(jnp.where(e_cols[k] == iota_row, avail, 0.0),
                      axis=1, keepdims=True)
        pos_k = b_k + within[k]
        keep = pos_k < C
        slot_cols.append(jnp.where(keep, e_cols[k] * C + pos_k, 0.0))
        wf_cols.append(jnp.where(keep, wn[:, k:k + 1], 0.0))
    pad = jnp.zeros((TB, 2), jnp.float32)
    slot8 = jnp.concatenate(slot_cols + [pad], axis=1)
    slot_ref[...] = slot8.astype(jnp.int32)
    slott_ref[...] = jnp.transpose(slot8, (1, 0)).astype(jnp.int32)
    wrep = [jnp.broadcast_to(wf_cols[k], (TB, 16)) for k in range(K)]
    wrep.append(jnp.zeros((TB, 32), jnp.float32))
    w_ref[...] = jnp.concatenate(wrep, axis=1)

    new_base = base + jnp.sum(oh, axis=0, keepdims=True)
    cnt_scr[...] = jnp.broadcast_to(new_base, (8, E))
    counts_ref[...] = jnp.broadcast_to(new_base, (8, E)).astype(jnp.int32)

    sg = lax.dot_general(xb, swg_ref[...], (((1,), (0,)), ((), ())),
                         precision=df, preferred_element_type=jnp.float32)
    su = lax.dot_general(xb, swu_ref[...], (((1,), (0,)), ((), ())),
                         precision=df, preferred_element_type=jnp.float32)
    sint = jnp.clip(_silu(sg) * su, -LIMIT, LIMIT)
    shared_ref[...] = lax.dot_general(sint, swd_ref[...],
                                      (((1,), (0,)), ((), ())),
                                      precision=df,
                                      preferred_element_type=jnp.float32)


def _router_call(x, gate_w, bias2, sw_gate, sw_up, sw_down):
    grid = (T // TB,)
    return pl.pallas_call(
        _router_body,
        grid=grid,
        in_specs=[
            pl.BlockSpec((TB, H), lambda i: (i, 0)),
            pl.BlockSpec((E, H), lambda i: (0, 0)),
            pl.BlockSpec((8, E), lambda i: (0, 0)),
            pl.BlockSpec((H, I_S), lambda i: (0, 0)),
            pl.BlockSpec((H, I_S), lambda i: (0, 0)),
            pl.BlockSpec((I_S, H), lambda i: (0, 0)),
        ],
        out_specs=[
            pl.BlockSpec((TB, H), lambda i: (i, 0)),
            pl.BlockSpec((TB, 8), lambda i: (i, 0)),
            pl.BlockSpec((8, TB), lambda i: (0, i)),
            pl.BlockSpec((TB, 128), lambda i: (i, 0)),
            pl.BlockSpec((8, E), lambda i: (0, 0)),
        ],
        out_shape=[
            jax.ShapeDtypeStruct((T, H), jnp.float32),
            jax.ShapeDtypeStruct((T, 8), jnp.int32),
            jax.ShapeDtypeStruct((8, T), jnp.int32),
            jax.ShapeDtypeStruct((T, 128), jnp.float32),
            jax.ShapeDtypeStruct((8, E), jnp.int32),
        ],
        scratch_shapes=[pltpu.VMEM((8, E), jnp.float32)],
    )(x, gate_w, bias2, sw_gate, sw_up, sw_down)


def _ffn_body(buf_ref, wg_ref, wu_ref, wd_ref, out_ref):
    df = jax.lax.Precision.DEFAULT
    rows = buf_ref[...]
    wg = wg_ref[0]
    wu = wu_ref[0]
    wd = wd_ref[0]
    gp = lax.dot_general(rows, wg, (((1,), (0,)), ((), ())),
                         precision=df, preferred_element_type=jnp.float32)
    up = lax.dot_general(rows, wu, (((1,), (0,)), ((), ())),
                         precision=df, preferred_element_type=jnp.float32)
    inter = jnp.clip(_silu(gp) * up, -LIMIT, LIMIT)
    out_ref[...] = lax.dot_general(inter, wd, (((1,), (0,)), ((), ())),
                                   precision=df,
                                   preferred_element_type=jnp.float32)


def _ffn_call(buf, w_gate, w_up, w_down):
    return pl.pallas_call(
        _ffn_body,
        grid=(E,),
        in_specs=[
            pl.BlockSpec((C, H), lambda e: (e, 0)),
            pl.BlockSpec((1, H, I_DIM), lambda e: (e, 0, 0)),
            pl.BlockSpec((1, H, I_DIM), lambda e: (e, 0, 0)),
            pl.BlockSpec((1, I_DIM, H), lambda e: (e, 0, 0)),
        ],
        out_specs=pl.BlockSpec((C, H), lambda e: (e, 0)),
        out_shape=jax.ShapeDtypeStruct((E * C, H), jnp.float32),
    )(buf, w_gate, w_up, w_down)


DTCH = 32


def _dispatch_body(x_hbm, slott_hbm, buf_hbm,
                   rows, idxs, sem_x, sem_s):
    wid = lax.axis_index("s") * 2 + lax.axis_index("c")
    tbase = wid * TPW
    for c2 in range(TPW // DTCH):
        tb = tbase + c2 * DTCH
        pltpu.sync_copy(x_hbm.at[pl.ds(tb, DTCH)], rows)
        for k in range(K):
            pltpu.sync_copy(slott_hbm.at[k, pl.ds(tb, DTCH)], idxs[k])
        cps = [pltpu.async_copy(rows, buf_hbm.at[idxs[k]], sem_s)
               for k in range(K)]
        for cp in cps:
            cp.wait()


def _dispatch_call(x, slott):
    mesh = plsc.VectorSubcoreMesh(core_axis_name="c", subcore_axis_name="s", num_cores=2, num_subcores=16)
    kern = pl.kernel(
        _dispatch_body,
        out_type=jax.ShapeDtypeStruct((E * C, H), jnp.float32),
        mesh=mesh,
        scratch_types=[
            pltpu.VMEM((DTCH, H), jnp.float32),
            [pltpu.VMEM((DTCH,), jnp.int32) for _ in range(K)],
            pltpu.SemaphoreType.DMA,
            pltpu.SemaphoreType.DMA,
        ],
    )
    return kern(x, slott)


CCH = 8
CPCH = CCH * K


def _combine_body(eout_hbm, slot_hbm, w_hbm, shared_hbm, out_hbm,
                  slotv, wv, rows2, acc, sem2):
    wid = lax.axis_index("s") * 2 + lax.axis_index("c")
    nch = TPW // CCH

    def start_gather(c):
        b = c % 2
        pltpu.sync_copy(slot_hbm.at[pl.ds(wid * PPW + c * CPCH, CPCH)],
                        slotv[b])
        return pltpu.async_copy(eout_hbm.at[slotv[b]], rows2[b], sem2[b])

    cp = start_gather(0)
    for c4 in range(nch):
        nxt = start_gather(c4 + 1) if c4 + 1 < nch else None
        cp.wait()
        b = c4 % 2
        rows = rows2[b]
        tb = wid * TPW + c4 * CCH
        pltpu.sync_copy(w_hbm.at[pl.ds(tb, CCH)], wv)
        pltpu.sync_copy(shared_hbm.at[pl.ds(tb, CCH)], acc)
        for tk in range(CCH):
            wspl = [wv[tk, pl.ds(k * 16, 16)] for k in range(K)]

            @plsc.parallel_loop(0, H // 16, 1, unroll=4)
            def _col_body(c, tk=tk, wspl=wspl, rows=rows):
                a = acc[tk, pl.ds(c * 16, 16)]
                for k in range(K):
                    r = rows[tk * K + k, pl.ds(c * 16, 16)]
                    contrib = jnp.where(wspl[k] != 0.0, wspl[k] * r, 0.0)
                    a = a + contrib
                acc[tk, pl.ds(c * 16, 16)] = a
        pltpu.sync_copy(acc, out_hbm.at[pl.ds(tb, CCH)])
        cp = nxt


def _combine_call(eout, slot_flat, w_rep, shared):
    mesh = plsc.VectorSubcoreMesh(core_axis_name="c", subcore_axis_name="s", num_cores=2, num_subcores=16)
    kern = pl.kernel(
        _combine_body,
        out_type=jax.ShapeDtypeStruct((T, H), jnp.float32),
        mesh=mesh,
        scratch_types=[
            [pltpu.VMEM((CPCH,), jnp.int32) for _ in range(2)],
            pltpu.VMEM((CCH, 128), jnp.float32),
            [pltpu.VMEM((CPCH, H), jnp.float32) for _ in range(2)],
            pltpu.VMEM((CCH, H), jnp.float32),
            [pltpu.SemaphoreType.DMA for _ in range(2)],
        ],
    )
    return kern(eout, slot_flat, w_rep, shared)


def kernel(x, gate_w, bias, w_gate, w_up, w_down, sw_gate, sw_up, sw_down):
    bias2 = jnp.broadcast_to(bias.reshape(1, E), (8, E))
    shared, slot8, slott, w_rep, _counts = _router_call(
        x, gate_w, bias2, sw_gate, sw_up, sw_down)
    slot_flat = slot8[:, :K].reshape(-1)
    buf = _dispatch_call(x, slott)
    eout = _ffn_call(buf, w_gate, w_up, w_down)
    out = _combine_call(eout, slot_flat, w_rep, shared)
    return out


# speedup vs baseline: 1.6259x; 1.0371x over previous
import functools

import jax
import jax.numpy as jnp
import numpy as np
from jax import lax
from jax.experimental import pallas as pl
from jax.experimental.pallas import tpu as pltpu
from jax.experimental.pallas import tpu_sc as plsc

T = 2048
H = 1024
I_DIM = 512
E = 64
K = 6
G = 8
TOPK_G = 4
C = 384
SCALE = 1.5
LIMIT = 10.0
I_S = 512

TB = 256
NW = 32
PAIRS = T * K
PPW = PAIRS // NW
PCH = 96
TPW = T // NW
TCH = 16

_NEG_INF = float("-inf")


def _roll(a, r):
    r = r % E
    if r == 0:
        return a
    return jnp.concatenate([a[:, r:], a[:, :r]], axis=1)


def _group_butterfly(a, op, lane_i):
    for s in (1, 2, 4):
        m = (lane_i & s) == 0
        partner = jnp.where(m, _roll(a, s), _roll(a, -s))
        a = op(a, partner)
    return a


def _silu(v):
    return v * (1.0 / (1.0 + jnp.exp(-v)))


def _router_body(x_ref, gw_ref, bias_ref, swg_ref, swu_ref, swd_ref,
                 shared_ref, slot_ref, slott_ref, w_ref, counts_ref,
                 cnt_scr):
    pid = pl.program_id(0)

    @pl.when(pid == 0)
    def _():
        cnt_scr[...] = jnp.zeros((8, E), jnp.float32)

    xb = x_ref[...]
    hi = jax.lax.Precision.HIGHEST
    df = jax.lax.Precision.DEFAULT
    logits = lax.dot_general(xb, gw_ref[...], (((1,), (1,)), ((), ())),
                             precision=df, preferred_element_type=jnp.float32)
    sp = jnp.maximum(logits, 0.0) + jnp.log1p(jnp.exp(-jnp.abs(logits)))
    scores = jnp.sqrt(sp)
    s4c = scores + bias_ref[0:1, :]

    lane_f = lax.broadcasted_iota(jnp.int32, (TB, E), 1).astype(jnp.float32)
    lane_i = lax.broadcasted_iota(jnp.int32, (1, E), 1)

    pmax = jnp.full((TB, E), _NEG_INF, jnp.float32)
    for r in range(1, E // G):
        cand = s4c + _roll(s4c, r)
        ok = (lane_i & 7) < (8 - r)
        pmax = jnp.maximum(pmax, jnp.where(ok, cand, _NEG_INF))
    g2 = _group_butterfly(pmax, jnp.maximum, lane_i)

    gid_i = lane_i // (E // G)
    gid_f = gid_i.astype(jnp.float32)
    rank = jnp.zeros((TB, E), jnp.float32)
    for m in range(1, G):
        sj = _roll(g2, 8 * m)
        j_f = ((gid_i + m) % G).astype(jnp.float32)
        beats = (sj > g2) | ((sj == g2) & (j_f < gid_f))
        rank = rank + beats.astype(jnp.float32)
    masked = jnp.where(rank < TOPK_G, s4c, _NEG_INF)

    cur = masked
    idx_cols, w_cols = [], []
    for _k in range(K):
        mval = jnp.max(cur, axis=1, keepdims=True)
        cnd = jnp.where(cur == mval, lane_f, 1e9)
        am = jnp.min(cnd, axis=1, keepdims=True)
        sel = lane_f == am
        w_cols.append(jnp.sum(jnp.where(sel, scores, 0.0), axis=1,
                              keepdims=True))
        idx_cols.append(am)
        cur = jnp.where(sel, _NEG_INF, cur)
    idxs = jnp.concatenate(idx_cols, axis=1)
    ws = jnp.concatenate(w_cols, axis=1)
    wn = ws / (jnp.sum(ws, axis=1, keepdims=True) + 1e-20) * SCALE

    e_cols = [idxs[:, j:j + 1] for j in range(K)]
    iota_row = lane_i.astype(jnp.float32)
    oh = jnp.zeros((TB, E), jnp.float32)
    for j in range(K):
        oh = oh + (e_cols[j] == iota_row).astype(jnp.float32)
    r_i = lax.broadcasted_iota(jnp.int32, (TB, TB), 0)
    c_j = lax.broadcasted_iota(jnp.int32, (TB, TB), 1)
    ltri = (c_j < r_i).astype(jnp.float32)
    rowcum = lax.dot_general(ltri, oh, (((1,), (0,)), ((), ())),
                             precision=hi, preferred_element_type=jnp.float32)
    base = cnt_scr[0:1, :]
    avail = base + rowcum

    slot_cols, wf_cols = [], []
    within = [jnp.zeros((TB, 1), jnp.float32) for _ in range(K)]
    for k in range(K):
        for j in range(k):
            within[k] = within[k] + (e_cols[j] == e_cols[k]).astype(jnp.float32)
        b_k = jnp.sum(jnp.where(e_cols[k] == iota_row, avail, 0.0),
                      axis=1, keepdims=True)
        pos_k = b_k + within[k]
        keep = pos_k < C
        slot_cols.append(jnp.where(keep, e_cols[k] * C + pos_k, 0.0))
        wf_cols.append(jnp.where(keep, wn[:, k:k + 1], 0.0))
    pad = jnp.zeros((TB, 2), jnp.float32)
    slot8 = jnp.concatenate(slot_cols + [pad], axis=1)
    slot_ref[...] = slot8.astype(jnp.int32)
    slott_ref[...] = jnp.transpose(slot8, (1, 0)).astype(jnp.int32)
    wrep = [jnp.broadcast_to(wf_cols[k], (TB, 16)) for k in range(K)]
    wrep.append(jnp.zeros((TB, 32), jnp.float32))
    w_ref[...] = jnp.concatenate(wrep, axis=1)

    new_base = base + jnp.sum(oh, axis=0, keepdims=True)
    cnt_scr[...] = jnp.broadcast_to(new_base, (8, E))
    counts_ref[...] = jnp.broadcast_to(new_base, (8, E)).astype(jnp.int32)

    sg = lax.dot_general(xb, swg_ref[...], (((1,), (0,)), ((), ())),
                         precision=df, preferred_element_type=jnp.float32)
    su = lax.dot_general(xb, swu_ref[...], (((1,), (0,)), ((), ())),
                         precision=df, preferred_element_type=jnp.float32)
    sint = jnp.clip(_silu(sg) * su, -LIMIT, LIMIT)
    shared_ref[...] = lax.dot_general(sint, swd_ref[...],
                                      (((1,), (0,)), ((), ())),
                                      precision=df,
                                      preferred_element_type=jnp.float32)


def _router_call(x, gate_w, bias2, sw_gate, sw_up, sw_down):
    grid = (T // TB,)
    return pl.pallas_call(
        _router_body,
        grid=grid,
        in_specs=[
            pl.BlockSpec((TB, H), lambda i: (i, 0)),
            pl.BlockSpec((E, H), lambda i: (0, 0)),
            pl.BlockSpec((8, E), lambda i: (0, 0)),
            pl.BlockSpec((H, I_S), lambda i: (0, 0)),
            pl.BlockSpec((H, I_S), lambda i: (0, 0)),
            pl.BlockSpec((I_S, H), lambda i: (0, 0)),
        ],
        out_specs=[
            pl.BlockSpec((TB, H), lambda i: (i, 0)),
            pl.BlockSpec((TB, 8), lambda i: (i, 0)),
            pl.BlockSpec((8, TB), lambda i: (0, i)),
            pl.BlockSpec((TB, 128), lambda i: (i, 0)),
            pl.BlockSpec((8, E), lambda i: (0, 0)),
        ],
        out_shape=[
            jax.ShapeDtypeStruct((T, H), jnp.float32),
            jax.ShapeDtypeStruct((T, 8), jnp.int32),
            jax.ShapeDtypeStruct((8, T), jnp.int32),
            jax.ShapeDtypeStruct((T, 128), jnp.float32),
            jax.ShapeDtypeStruct((8, E), jnp.int32),
        ],
        scratch_shapes=[pltpu.VMEM((8, E), jnp.float32)],
    )(x, gate_w, bias2, sw_gate, sw_up, sw_down)


def _ffn_body(buf_ref, wg_ref, wu_ref, wd_ref, out_ref):
    df = jax.lax.Precision.DEFAULT
    rows = buf_ref[...]
    wg = wg_ref[0]
    wu = wu_ref[0]
    wd = wd_ref[0]
    gp = lax.dot_general(rows, wg, (((1,), (0,)), ((), ())),
                         precision=df, preferred_element_type=jnp.float32)
    up = lax.dot_general(rows, wu, (((1,), (0,)), ((), ())),
                         precision=df, preferred_element_type=jnp.float32)
    inter = jnp.clip(_silu(gp) * up, -LIMIT, LIMIT)
    out_ref[...] = lax.dot_general(inter, wd, (((1,), (0,)), ((), ())),
                                   precision=df,
                                   preferred_element_type=jnp.float32)


def _ffn_call(buf, w_gate, w_up, w_down):
    return pl.pallas_call(
        _ffn_body,
        grid=(E,),
        in_specs=[
            pl.BlockSpec((C, H), lambda e: (e, 0)),
            pl.BlockSpec((1, H, I_DIM), lambda e: (e, 0, 0)),
            pl.BlockSpec((1, H, I_DIM), lambda e: (e, 0, 0)),
            pl.BlockSpec((1, I_DIM, H), lambda e: (e, 0, 0)),
        ],
        out_specs=pl.BlockSpec((C, H), lambda e: (e, 0)),
        out_shape=jax.ShapeDtypeStruct((E * C, H), jnp.float32),
    )(buf, w_gate, w_up, w_down)


DTCH = 32


def _dispatch_body(x_hbm, slott_hbm, buf_hbm,
                   rows, idxs, sem_x, sem_s):
    wid = lax.axis_index("s") * 2 + lax.axis_index("c")
    tbase = wid * TPW
    for c2 in range(TPW // DTCH):
        tb = tbase + c2 * DTCH
        pltpu.sync_copy(x_hbm.at[pl.ds(tb, DTCH)], rows)
        for k in range(K):
            pltpu.sync_copy(slott_hbm.at[k, pl.ds(tb, DTCH)], idxs[k])
        cps = [pltpu.async_copy(rows, buf_hbm.at[idxs[k]], sem_s)
               for k in range(K)]
        for cp in cps:
            cp.wait()


def _dispatch_call(x, slott):
    mesh = plsc.VectorSubcoreMesh(core_axis_name="c", subcore_axis_name="s", num_cores=2, num_subcores=16)
    kern = pl.kernel(
        _dispatch_body,
        out_type=jax.ShapeDtypeStruct((E * C, H), jnp.float32),
        mesh=mesh,
        scratch_types=[
            pltpu.VMEM((DTCH, H), jnp.float32),
            [pltpu.VMEM((DTCH,), jnp.int32) for _ in range(K)],
            pltpu.SemaphoreType.DMA,
            pltpu.SemaphoreType.DMA,
        ],
    )
    return kern(x, slott)


CCH = 8
CPCH = CCH * K


def _combine_body(eout_hbm, slot_hbm, w_hbm, shared_hbm, out_hbm,
                  slotv, wv, rows2, acc2, sem2, semsh, semo):
    wid = lax.axis_index("s") * 2 + lax.axis_index("c")
    nch = TPW // CCH
    pltpu.sync_copy(slot_hbm.at[pl.ds(wid * PPW, PPW)], slotv)
    pltpu.sync_copy(w_hbm.at[pl.ds(wid * TPW, TPW)], wv)

    def start_gather(c):
        return pltpu.async_copy(eout_hbm.at[slotv.at[pl.ds(c * CPCH, CPCH)]],
                                rows2[c % 2], sem2[c % 2])

    def start_shared(c):
        tb = wid * TPW + c * CCH
        return pltpu.async_copy(shared_hbm.at[pl.ds(tb, CCH)],
                                acc2[c % 2], semsh[c % 2])

    cp = start_gather(0)
    sp = start_shared(0)
    outcp = None
    for c4 in range(nch):
        nxt = start_gather(c4 + 1) if c4 + 1 < nch else None
        cp.wait()
        sp.wait()
        b = c4 % 2
        rows = rows2[b]
        acc = acc2[b]
        tb = wid * TPW + c4 * CCH
        for tk in range(CCH):
            wspl = [wv[c4 * CCH + tk, pl.ds(k * 16, 16)] for k in range(K)]

            @plsc.parallel_loop(0, H // 16, 1, unroll=4)
            def _col_body(c, tk=tk, wspl=wspl, rows=rows, acc=acc):
                a = acc[tk, pl.ds(c * 16, 16)]
                for k in range(K):
                    r = rows[tk * K + k, pl.ds(c * 16, 16)]
                    contrib = jnp.where(wspl[k] != 0.0, wspl[k] * r, 0.0)
                    a = a + contrib
                acc[tk, pl.ds(c * 16, 16)] = a
        if outcp is not None:
            outcp.wait()
        sp = start_shared(c4 + 1) if c4 + 1 < nch else None
        outcp = pltpu.async_copy(acc, out_hbm.at[pl.ds(tb, CCH)], semo)
        cp = nxt
    outcp.wait()


def _combine_call(eout, slot_flat, w_rep, shared):
    mesh = plsc.VectorSubcoreMesh(core_axis_name="c", subcore_axis_name="s", num_cores=2, num_subcores=16)
    kern = pl.kernel(
        _combine_body,
        out_type=jax.ShapeDtypeStruct((T, H), jnp.float32),
        mesh=mesh,
        scratch_types=[
            pltpu.VMEM((PPW,), jnp.int32),
            pltpu.VMEM((TPW, 128), jnp.float32),
            [pltpu.VMEM((CPCH, H), jnp.float32) for _ in range(2)],
            [pltpu.VMEM((CCH, H), jnp.float32) for _ in range(2)],
            [pltpu.SemaphoreType.DMA for _ in range(2)],
            [pltpu.SemaphoreType.DMA for _ in range(2)],
            pltpu.SemaphoreType.DMA,
        ],
    )
    return kern(eout, slot_flat, w_rep, shared)


def kernel(x, gate_w, bias, w_gate, w_up, w_down, sw_gate, sw_up, sw_down):
    bias2 = jnp.broadcast_to(bias.reshape(1, E), (8, E))
    shared, slot8, slott, w_rep, _counts = _router_call(
        x, gate_w, bias2, sw_gate, sw_up, sw_down)
    slot_flat = slot8[:, :K].reshape(-1)
    buf = _dispatch_call(x, slott)
    eout = _ffn_call(buf, w_gate, w_up, w_down)
    out = _combine_call(eout, slot_flat, w_rep, shared)
    return out
